# baseline (speedup 1.0000x reference)
import jax
import jax.numpy as jnp
from jax import lax
from jax.experimental import pallas as pl
from jax.experimental.pallas import tpu as pltpu
from jax.experimental.pallas import tpu_sc as plsc

NUM_LOCAL = 10000
NUM_HALO = 2000
NUM_EDGES = 320000
D = 128

NC = 2
NS = 16
NW = NC * NS
L = 16

PER_TILE = NUM_EDGES // NW
CHUNK = 128
NBUF = 3
NTRIP = PER_TILE // (NBUF * CHUNK)
TAIL = PER_TILE - NTRIP * NBUF * CHUNK

RTBL_ROWS = 12048

ROWS_PER_TILE = 632
ACC_ROWS = ROWS_PER_TILE * NS
SCRAP_ROW = NUM_LOCAL


def _sc_body(src_hbm, dst_hbm, rtbl_hbm, x_hbm, zero_hbm, agg_hbm,
             srcs, dsts, src_t, dst_t, rows, gsems, ssems, acc_sh):
    cid = lax.axis_index("c")
    sid = lax.axis_index("s")
    wid = sid * NC + cid

    pltpu.sync_copy(zero_hbm, rows[0])
    done = 0
    zdescs = []
    while done < ROWS_PER_TILE:
        c = min(CHUNK, ROWS_PER_TILE - done)
        zdescs.append(pltpu.async_copy(
            rows[0].at[pl.ds(0, c)],
            acc_sh.at[pl.ds(sid * ROWS_PER_TILE + done, c)],
            gsems[len(zdescs) % NBUF]))
        done += c
    for d in zdescs:
        d.wait()
    plsc.subcore_barrier()

    base = wid * PER_TILE

    def trip_body(k, carry):
        rdescs = []
        for j in range(NBUF):
            off = base + (k * NBUF + j) * CHUNK

            @pl.when(k > 0)
            def _(j=j):
                pltpu.make_async_copy(rows[j], acc_sh.at[srcs[j]],
                                      ssems[j]).wait()

            pltpu.sync_copy(src_hbm.at[pl.ds(off, CHUNK)], srcs[j])
            rdescs.append(pltpu.async_copy(rtbl_hbm.at[srcs[j]], dsts[j],
                                           gsems[j]))
        descs = []
        for j in range(NBUF):
            off = base + (k * NBUF + j) * CHUNK
            rdescs[j].wait()
            pltpu.sync_copy(dst_hbm.at[pl.ds(off, CHUNK)], srcs[j])
            descs.append(pltpu.async_copy(x_hbm.at[dsts[j]], rows[j],
                                          gsems[j]))
        for j in range(NBUF):
            descs[j].wait()
            pltpu.async_copy(rows[j], acc_sh.at[srcs[j]], ssems[j], add=True)
        return carry

    lax.fori_loop(0, NTRIP, trip_body, 0)
    for j in range(NBUF):
        pltpu.make_async_copy(rows[j], acc_sh.at[srcs[j]], ssems[j]).wait()

    offt = base + NTRIP * NBUF * CHUNK
    pltpu.sync_copy(src_hbm.at[pl.ds(offt, TAIL)], src_t)
    pltpu.async_copy(rtbl_hbm.at[src_t], dst_t, gsems[0]).wait()
    src_t[...] = dst_t[...]
    pltpu.sync_copy(dst_hbm.at[pl.ds(offt, TAIL)], dst_t)
    pltpu.async_copy(x_hbm.at[src_t], rows[0].at[pl.ds(0, TAIL)],
                     gsems[0]).wait()
    pltpu.sync_copy(rows[0].at[pl.ds(0, TAIL)], acc_sh.at[dst_t], add=True)
    plsc.subcore_barrier()

    a0 = sid * ROWS_PER_TILE
    done = 0
    outs = []
    while done < ROWS_PER_TILE:
        c = min(CHUNK, ROWS_PER_TILE - done)
        j = len(outs) % NBUF
        if len(outs) >= NBUF:
            outs[len(outs) - NBUF].wait()
        pltpu.sync_copy(acc_sh.at[pl.ds(a0 + done, c)], rows[j].at[pl.ds(0, c)])
        outs.append(pltpu.async_copy(rows[j].at[pl.ds(0, c)],
                                     agg_hbm.at[cid, pl.ds(a0 + done, c)],
                                     ssems[j]))
        done += c
    for d in outs[-NBUF:]:
        d.wait()


def _sc_message_pass(src, dst, rtbl, x, zero_init):
    mesh = plsc.VectorSubcoreMesh(core_axis_name="c", subcore_axis_name="s")
    return pl.kernel(
        _sc_body,
        out_type=jax.ShapeDtypeStruct((NC, ACC_ROWS, D), jnp.float32),
        mesh=mesh,
        scratch_types=[
            [pltpu.VMEM((CHUNK,), jnp.int32) for _ in range(NBUF)],
            [pltpu.VMEM((CHUNK,), jnp.int32) for _ in range(NBUF)],
            pltpu.VMEM((TAIL,), jnp.int32),
            pltpu.VMEM((TAIL,), jnp.int32),
            [pltpu.VMEM((CHUNK, D), jnp.float32) for _ in range(NBUF)],
            [pltpu.SemaphoreType.DMA for _ in range(NBUF)],
            [pltpu.SemaphoreType.DMA for _ in range(NBUF)],
            pltpu.VMEM_SHARED((ACC_ROWS, D), jnp.float32),
        ],
    )(src, dst, rtbl, x, zero_init)


def _tc_body(x_ref, a0_ref, a1_ref, ws_ref, wn_ref, b_ref, o_ref):
    agg = a0_ref[0] + a1_ref[0]
    o_ref[...] = (
        jnp.dot(x_ref[...], ws_ref[...], preferred_element_type=jnp.float32)
        + jnp.dot(agg, wn_ref[...], preferred_element_type=jnp.float32)
        + b_ref[...]
    )


def _tc_combine(x, aggp, W_self, W_neigh, b2):
    blk = 1000
    grid = (NUM_LOCAL // blk,)
    return pl.pallas_call(
        _tc_body,
        grid=grid,
        in_specs=[
            pl.BlockSpec((blk, D), lambda i: (i, 0)),
            pl.BlockSpec((1, blk, D), lambda i: (0, i, 0)),
            pl.BlockSpec((1, blk, D), lambda i: (1, i, 0)),
            pl.BlockSpec((D, D), lambda i: (0, 0)),
            pl.BlockSpec((D, D), lambda i: (0, 0)),
            pl.BlockSpec((1, D), lambda i: (0, 0)),
        ],
        out_specs=pl.BlockSpec((blk, D), lambda i: (i, 0)),
        out_shape=jax.ShapeDtypeStruct((NUM_LOCAL, D), jnp.float32),
    )(x, aggp, aggp, W_self, W_neigh, b2)


@jax.jit
def kernel(local_node_features, send_local_idx, edge_index, W_self, W_neigh, b):
    src = edge_index[:, 0].astype(jnp.int32)
    dst = jnp.minimum(edge_index[:, 1].astype(jnp.int32), SCRAP_ROW)
    rtbl = jnp.concatenate([
        jnp.arange(NUM_LOCAL, dtype=jnp.int32),
        send_local_idx.astype(jnp.int32),
        jnp.zeros((RTBL_ROWS - NUM_LOCAL - NUM_HALO,), jnp.int32),
    ])
    zero_init = jnp.zeros((CHUNK, D), jnp.float32)
    aggp = _sc_message_pass(src, dst, rtbl, local_node_features, zero_init)
    return _tc_combine(local_node_features, aggp, W_self, W_neigh,
                       b.reshape(1, D))


# speedup vs baseline: 1.0086x; 1.0086x over previous
import jax
import jax.numpy as jnp
from jax import lax
from jax.experimental import pallas as pl
from jax.experimental.pallas import tpu as pltpu
from jax.experimental.pallas import tpu_sc as plsc

NUM_LOCAL = 10000
NUM_HALO = 2000
NUM_EDGES = 320000
D = 128

NC = 2
NS = 16
NW = NC * NS
L = 16

PER_TILE = NUM_EDGES // NW
CHUNK = 128
NBUF = 3
NTRIP = PER_TILE // (NBUF * CHUNK)
TAIL = PER_TILE - NTRIP * NBUF * CHUNK

RTBL_ROWS = 12048

ROWS_PER_TILE = 632
ACC_ROWS = ROWS_PER_TILE * NS
SCRAP_ROW = NUM_LOCAL


def _sc_body(src_hbm, dst_hbm, rtbl_hbm, x_hbm, zero_hbm, agg_hbm,
             srcs, dsts, src_t, dst_t, rows, gsems, ssems, acc_sh):
    cid = lax.axis_index("c")
    sid = lax.axis_index("s")
    wid = sid * NC + cid

    pltpu.sync_copy(zero_hbm, rows[0])
    done = 0
    zdescs = []
    while done < ROWS_PER_TILE:
        c = min(CHUNK, ROWS_PER_TILE - done)
        zdescs.append(pltpu.async_copy(
            rows[0].at[pl.ds(0, c)],
            acc_sh.at[pl.ds(sid * ROWS_PER_TILE + done, c)],
            gsems[len(zdescs) % NBUF]))
        done += c
    for d in zdescs:
        d.wait()
    plsc.subcore_barrier()

    base = wid * PER_TILE

    def clamp(dv):
        for i in range(CHUNK // L):
            dv[pl.ds(i * L, L)] = jnp.minimum(dv[pl.ds(i * L, L)], SCRAP_ROW)

    def trip_body(k, carry):
        rdescs = []
        for j in range(NBUF):
            off = base + (k * NBUF + j) * CHUNK

            @pl.when(k > 0)
            def _(j=j):
                pltpu.make_async_copy(rows[j], acc_sh.at[srcs[j]],
                                      ssems[j]).wait()

            pltpu.sync_copy(src_hbm.at[pl.ds(off, CHUNK)], srcs[j])
            rdescs.append(pltpu.async_copy(rtbl_hbm.at[srcs[j]], dsts[j],
                                           gsems[j]))
        descs = []
        for j in range(NBUF):
            off = base + (k * NBUF + j) * CHUNK
            rdescs[j].wait()
            pltpu.sync_copy(dst_hbm.at[pl.ds(off, CHUNK)], srcs[j])
            clamp(srcs[j])
            descs.append(pltpu.async_copy(x_hbm.at[dsts[j]], rows[j],
                                          gsems[j]))
        for j in range(NBUF):
            descs[j].wait()
            pltpu.async_copy(rows[j], acc_sh.at[srcs[j]], ssems[j], add=True)
        return carry

    lax.fori_loop(0, NTRIP, trip_body, 0)
    for j in range(NBUF):
        pltpu.make_async_copy(rows[j], acc_sh.at[srcs[j]], ssems[j]).wait()

    offt = base + NTRIP * NBUF * CHUNK
    pltpu.sync_copy(src_hbm.at[pl.ds(offt, TAIL)], src_t)
    pltpu.async_copy(rtbl_hbm.at[src_t], dst_t, gsems[0]).wait()
    src_t[...] = dst_t[...]
    pltpu.sync_copy(dst_hbm.at[pl.ds(offt, TAIL)], dst_t)
    dst_t[...] = jnp.minimum(dst_t[...], SCRAP_ROW)
    pltpu.async_copy(x_hbm.at[src_t], rows[0].at[pl.ds(0, TAIL)],
                     gsems[0]).wait()
    pltpu.sync_copy(rows[0].at[pl.ds(0, TAIL)], acc_sh.at[dst_t], add=True)
    plsc.subcore_barrier()

    a0 = sid * ROWS_PER_TILE
    done = 0
    outs = []
    while done < ROWS_PER_TILE:
        c = min(CHUNK, ROWS_PER_TILE - done)
        j = len(outs) % NBUF
        if len(outs) >= NBUF:
            outs[len(outs) - NBUF].wait()
        pltpu.sync_copy(acc_sh.at[pl.ds(a0 + done, c)], rows[j].at[pl.ds(0, c)])
        outs.append(pltpu.async_copy(rows[j].at[pl.ds(0, c)],
                                     agg_hbm.at[cid, pl.ds(a0 + done, c)],
                                     ssems[j]))
        done += c
    for d in outs[-NBUF:]:
        d.wait()


def _sc_message_pass(src, dst, rtbl, x, zero_init):
    mesh = plsc.VectorSubcoreMesh(core_axis_name="c", subcore_axis_name="s")
    return pl.kernel(
        _sc_body,
        out_type=jax.ShapeDtypeStruct((NC, ACC_ROWS, D), jnp.float32),
        mesh=mesh,
        scratch_types=[
            [pltpu.VMEM((CHUNK,), jnp.int32) for _ in range(NBUF)],
            [pltpu.VMEM((CHUNK,), jnp.int32) for _ in range(NBUF)],
            pltpu.VMEM((TAIL,), jnp.int32),
            pltpu.VMEM((TAIL,), jnp.int32),
            [pltpu.VMEM((CHUNK, D), jnp.float32) for _ in range(NBUF)],
            [pltpu.SemaphoreType.DMA for _ in range(NBUF)],
            [pltpu.SemaphoreType.DMA for _ in range(NBUF)],
            pltpu.VMEM_SHARED((ACC_ROWS, D), jnp.float32),
        ],
    )(src, dst, rtbl, x, zero_init)


def _tc_body(x_ref, a0_ref, a1_ref, ws_ref, wn_ref, b_ref, o_ref):
    agg = a0_ref[0] + a1_ref[0]
    o_ref[...] = (
        jnp.dot(x_ref[...], ws_ref[...], preferred_element_type=jnp.float32)
        + jnp.dot(agg, wn_ref[...], preferred_element_type=jnp.float32)
        + b_ref[...]
    )


def _tc_combine(x, aggp, W_self, W_neigh, b2):
    blk = 1000
    grid = (NUM_LOCAL // blk,)
    return pl.pallas_call(
        _tc_body,
        grid=grid,
        in_specs=[
            pl.BlockSpec((blk, D), lambda i: (i, 0)),
            pl.BlockSpec((1, blk, D), lambda i: (0, i, 0)),
            pl.BlockSpec((1, blk, D), lambda i: (1, i, 0)),
            pl.BlockSpec((D, D), lambda i: (0, 0)),
            pl.BlockSpec((D, D), lambda i: (0, 0)),
            pl.BlockSpec((1, D), lambda i: (0, 0)),
        ],
        out_specs=pl.BlockSpec((blk, D), lambda i: (i, 0)),
        out_shape=jax.ShapeDtypeStruct((NUM_LOCAL, D), jnp.float32),
    )(x, aggp, aggp, W_self, W_neigh, b2)


@jax.jit
def kernel(local_node_features, send_local_idx, edge_index, W_self, W_neigh, b):
    src = edge_index[:, 0].astype(jnp.int32)
    dst = edge_index[:, 1].astype(jnp.int32)
    rtbl = jnp.concatenate([
        jnp.arange(NUM_LOCAL, dtype=jnp.int32),
        send_local_idx.astype(jnp.int32),
        jnp.zeros((RTBL_ROWS - NUM_LOCAL - NUM_HALO,), jnp.int32),
    ])
    zero_init = jnp.zeros((CHUNK, D), jnp.float32)
    aggp = _sc_message_pass(src, dst, rtbl, local_node_features, zero_init)
    return _tc_combine(local_node_features, aggp, W_self, W_neigh,
                       b.reshape(1, D))
